# baseline (speedup 1.0000x reference)
import functools

import jax
import jax.numpy as jnp
from jax import lax
from jax.experimental import pallas as pl
from jax.experimental.pallas import tpu as pltpu
from jax.experimental.pallas import tpu_sc as plsc

B, K, D = 16384, 20, 16
NH = D // 2
B_ET = 1000000
B_UT = 100000
GW = 8 * D

_info = plsc.get_sparse_core_info()
NC, NS, L = _info.num_cores, _info.num_subcores, _info.num_lanes
NW = NC * NS
BPW = B // NW
RC = 16
NCHUNK = BPW // RC


def _renorm_scale(ss):
    x = jnp.maximum(ss, 1e-24)
    xi = plsc.bitcast(x, jnp.int32)
    y = plsc.bitcast(jnp.int32(0x5F3759DF) - (xi >> 1), jnp.float32)
    for _ in range(3):
        y = y * (1.5 - 0.5 * x * y * y)
    return jnp.minimum(y, 1.0)


def _body(u_hbm, i_hbm, nb_hbm, ut_hbm, et_hbm, wt_hbm, a_hbm, out_hbm,
          i_v, u_v, nb_v, i8_v, u8_v, nb8_v, t_a, n_a, u_a, t_b, n_b, u_b,
          wt_v, a_v, wa0_v, wa1_v, out_v, sem_p, sem_a, sem_b):
    wid = lax.axis_index("s") * NC + lax.axis_index("c")
    base = wid * BPW

    pcps = [pltpu.async_copy(i_hbm.at[pl.ds(base, BPW)], i_v, sem_p),
            pltpu.async_copy(u_hbm.at[pl.ds(base, BPW)], u_v, sem_p),
            pltpu.async_copy(wt_hbm, wt_v, sem_p),
            pltpu.async_copy(a_hbm, a_v, sem_p)]
    for k in range(K):
        pcps.append(pltpu.async_copy(
            nb_hbm.at[pl.ds(k * B + base, BPW)],
            nb_v.at[pl.ds(k * BPW, BPW)], sem_p))
    for cp in pcps:
        cp.wait()

    iota = lax.iota(jnp.int32, L)

    def shift_into(src, dst, n):
        def lp(t, carry):
            dst[pl.ds(t * L, L)] = src[pl.ds(t * L, L)] >> 3
            return carry
        lax.fori_loop(0, n // L, lp, 0)
    shift_into(i_v, i8_v, BPW)
    shift_into(u_v, u8_v, BPW)
    shift_into(nb_v, nb8_v, BPW * K)

    avec = a_v[...]
    wa0 = jnp.zeros((L,), jnp.float32)
    wa1 = jnp.zeros((L,), jnp.float32)
    for j in range(NH):
        colj = wt_v[pl.ds(j * L, L)]
        wa0 = wa0 + colj * avec[j]
        wa1 = wa1 + colj * avec[NH + j]
    wa0_v[...] = wa0
    wa1_v[...] = wa1

    def fire(c, t_buf, n_buf, u_buf, sem):
        cb = c * RC
        pltpu.async_copy(et_hbm.at[i8_v.at[pl.ds(cb, RC)]], t_buf, sem)
        pltpu.async_copy(ut_hbm.at[u8_v.at[pl.ds(cb, RC)]], u_buf, sem)
        for k in range(K):
            pltpu.async_copy(
                et_hbm.at[nb8_v.at[pl.ds(k * BPW + cb, RC)]],
                n_buf.at[pl.ds(k * RC, RC)], sem)

    def drain(t_buf, n_buf, u_buf, sem):
        pltpu.make_async_copy(et_hbm.at[i8_v.at[pl.ds(0, RC)]], t_buf, sem).wait()
        pltpu.make_async_copy(ut_hbm.at[u8_v.at[pl.ds(0, RC)]], u_buf, sem).wait()
        for k in range(K):
            pltpu.make_async_copy(
                et_hbm.at[nb8_v.at[pl.ds(0, RC)]],
                n_buf.at[pl.ds(k * RC, RC)], sem).wait()

    def compute(c, t_buf, n_buf, u_buf):
        cb = c * RC
        wa0vec = wa0_v[...]
        wa1vec = wa1_v[...]
        wrows = [wt_v[pl.ds(j * L, L)] for j in range(NH)]

        off_t = (i_v[pl.ds(cb, L)] & 7) << 4
        off_u = (u_v[pl.ds(cb, L)] & 7) << 4
        off_n = [(nb_v[pl.ds(k * BPW + cb, L)] & 7) << 4 for k in range(K)]

        ss_t = jnp.zeros((L,), jnp.float32)
        dot_t = jnp.zeros((L,), jnp.float32)
        for d in range(D):
            col = plsc.load_gather(t_buf, [iota, off_t + d])
            ss_t += col * col
            dot_t += col * wa0vec[d]
        ts = dot_t * _renorm_scale(ss_t)

        m = None
        S = None
        agg = [None] * D
        for k in range(K):
            cols = []
            ss = jnp.zeros((L,), jnp.float32)
            dt = jnp.zeros((L,), jnp.float32)
            nrow = k * RC + iota
            for d in range(D):
                col = plsc.load_gather(n_buf, [nrow, off_n[k] + d])
                cols.append(col)
                ss += col * col
                dt += col * wa1vec[d]
            sc = _renorm_scale(ss)
            e = ts + dt * sc
            e = jnp.maximum(e, 0.2 * e)
            if m is None:
                m = e
                S = jnp.full((L,), 1.0, jnp.float32)
                for d in range(D):
                    agg[d] = sc * cols[d]
            else:
                m_new = jnp.maximum(m, e)
                r = jnp.exp(m - m_new)
                p = jnp.exp(e - m_new)
                S = S * r + p
                w = p * sc
                for d in range(D):
                    agg[d] = agg[d] * r + w * cols[d]
                m = m_new

        gj = [jnp.zeros((L,), jnp.float32) for _ in range(NH)]
        for d in range(D):
            for j in range(NH):
                gj[j] = gj[j] + agg[d] * wrows[j][d]

        ss_u = jnp.zeros((L,), jnp.float32)
        ucols = []
        for d in range(D):
            col = plsc.load_gather(u_buf, [iota, off_u + d])
            ss_u += col * col
            ucols.append(col)
        su = _renorm_scale(ss_u)
        uv = jnp.zeros((L,), jnp.float32)
        for j in range(NH):
            uv = uv + (ucols[j] + ucols[NH + j]) * gj[j]
        uv = uv * (su / S)
        out = 1.0 / (1.0 + jnp.exp(-uv))
        out_v[pl.ds(cb, L)] = out

    fire(0, t_a, n_a, u_a, sem_a)

    def pair_body(t, carry):
        c0 = 2 * t
        c1 = c0 + 1
        fire(c1, t_b, n_b, u_b, sem_b)
        drain(t_a, n_a, u_a, sem_a)
        compute(c0, t_a, n_a, u_a)

        @pl.when(c1 + 1 < NCHUNK)
        def _():
            fire(c1 + 1, t_a, n_a, u_a, sem_a)

        drain(t_b, n_b, u_b, sem_b)
        compute(c1, t_b, n_b, u_b)
        return carry

    lax.fori_loop(0, NCHUNK // 2, pair_body, 0)
    pltpu.sync_copy(out_v, out_hbm.at[pl.ds(base, BPW)])


_sc_call = functools.partial(
    pl.kernel,
    out_type=jax.ShapeDtypeStruct((B,), jnp.float32),
    mesh=plsc.VectorSubcoreMesh(core_axis_name="c", subcore_axis_name="s"),
    compiler_params=pltpu.CompilerParams(
        needs_layout_passes=False, use_tc_tiling_on_sc=True),
    scratch_types=[
        pltpu.VMEM((BPW,), jnp.int32),
        pltpu.VMEM((BPW,), jnp.int32),
        pltpu.VMEM((BPW * K,), jnp.int32),
        pltpu.VMEM((BPW,), jnp.int32),
        pltpu.VMEM((BPW,), jnp.int32),
        pltpu.VMEM((BPW * K,), jnp.int32),
        pltpu.VMEM((RC, GW), jnp.float32),
        pltpu.VMEM((RC * K, GW), jnp.float32),
        pltpu.VMEM((RC, GW), jnp.float32),
        pltpu.VMEM((RC, GW), jnp.float32),
        pltpu.VMEM((RC * K, GW), jnp.float32),
        pltpu.VMEM((RC, GW), jnp.float32),
        pltpu.VMEM((NH * L,), jnp.float32),
        pltpu.VMEM((L,), jnp.float32),
        pltpu.VMEM((D,), jnp.float32),
        pltpu.VMEM((D,), jnp.float32),
        pltpu.VMEM((BPW,), jnp.float32),
        pltpu.SemaphoreType.DMA,
        pltpu.SemaphoreType.DMA,
        pltpu.SemaphoreType.DMA,
    ],
)(_body)


def kernel(u, i, neighbors, user_table, entity_table, W, a):
    u = u.astype(jnp.int32)
    i = i.astype(jnp.int32)
    nb = neighbors.astype(jnp.int32).T.reshape(-1)
    wt = W.T.reshape(-1)
    return _sc_call(u, i, nb, user_table.reshape(-1, GW),
                    entity_table.reshape(-1, GW), wt, a.reshape(-1))


# speedup vs baseline: 1.1103x; 1.1103x over previous
import functools

import jax
import jax.numpy as jnp
from jax import lax
from jax.experimental import pallas as pl
from jax.experimental.pallas import tpu as pltpu
from jax.experimental.pallas import tpu_sc as plsc

B, K, D = 16384, 20, 16
NH = D // 2
B_ET = 1000000
B_UT = 100000

_info = plsc.get_sparse_core_info()
NC, NS, L = _info.num_cores, _info.num_subcores, _info.num_lanes
NW = NC * NS
BPW = B // NW
RC = 128
NCHUNK = BPW // RC
NG = RC // L


def _renorm_scale(ss):
    x = jnp.maximum(ss, 1e-24)
    xi = plsc.bitcast(x, jnp.int32)
    y = plsc.bitcast(jnp.int32(0x5F3759DF) - (xi >> 1), jnp.float32)
    for _ in range(3):
        y = y * (1.5 - 0.5 * x * y * y)
    return jnp.minimum(y, 1.0)


def _body(u_hbm, i_hbm, nb_hbm, ut_hbm, et_hbm, wt_hbm, a_hbm, out_hbm,
          i_v, u_v, nb_v, t_a, n_a, u_a, t_b, n_b, u_b,
          wt_v, a_v, wa0_v, wa1_v, out_v, sem_p, sem_a, sem_b):
    wid = lax.axis_index("s") * NC + lax.axis_index("c")
    base = wid * BPW

    pcps = [pltpu.async_copy(i_hbm.at[pl.ds(base, BPW)], i_v, sem_p),
            pltpu.async_copy(u_hbm.at[pl.ds(base, BPW)], u_v, sem_p),
            pltpu.async_copy(wt_hbm, wt_v, sem_p),
            pltpu.async_copy(a_hbm, a_v, sem_p)]
    for k in range(K):
        pcps.append(pltpu.async_copy(
            nb_hbm.at[pl.ds(k * B + base, BPW)],
            nb_v.at[pl.ds(k * BPW, BPW)], sem_p))
    for cp in pcps:
        cp.wait()

    iota = lax.iota(jnp.int32, L)

    avec = a_v[...]
    wa0 = jnp.zeros((L,), jnp.float32)
    wa1 = jnp.zeros((L,), jnp.float32)
    for j in range(NH):
        colj = wt_v[pl.ds(j * L, L)]
        wa0 = wa0 + colj * avec[j]
        wa1 = wa1 + colj * avec[NH + j]
    wa0_v[...] = wa0
    wa1_v[...] = wa1

    def fire(c, t_buf, n_buf, u_buf, sem):
        cb = c * RC
        pltpu.async_copy(et_hbm.at[i_v.at[pl.ds(cb, RC)]], t_buf, sem)
        pltpu.async_copy(ut_hbm.at[u_v.at[pl.ds(cb, RC)]], u_buf, sem)
        for k in range(K):
            pltpu.async_copy(
                et_hbm.at[nb_v.at[pl.ds(k * BPW + cb, RC)]],
                n_buf.at[pl.ds(k * RC, RC)], sem)

    def drain(t_buf, n_buf, u_buf, sem):
        pltpu.make_async_copy(et_hbm.at[i_v.at[pl.ds(0, RC)]], t_buf, sem).wait()
        pltpu.make_async_copy(ut_hbm.at[u_v.at[pl.ds(0, RC)]], u_buf, sem).wait()
        for k in range(K):
            pltpu.make_async_copy(
                et_hbm.at[nb_v.at[pl.ds(0, RC)]],
                n_buf.at[pl.ds(k * RC, RC)], sem).wait()

    def compute(c, t_buf, n_buf, u_buf):
        def group(g, carry2):
            rows = g * L + iota
            wa0vec = wa0_v[...]
            wa1vec = wa1_v[...]
            wrows = [wt_v[pl.ds(j * L, L)] for j in range(NH)]

            ss_t = jnp.zeros((L,), jnp.float32)
            dot_t = jnp.zeros((L,), jnp.float32)
            for d in range(D):
                col = plsc.load_gather(
                    t_buf, [rows, jnp.full((L,), d, jnp.int32)])
                ss_t += col * col
                dot_t += col * wa0vec[d]
            ts = dot_t * _renorm_scale(ss_t)

            m = None
            S = None
            agg = [None] * D
            for k in range(K):
                cols = []
                ss = jnp.zeros((L,), jnp.float32)
                dt = jnp.zeros((L,), jnp.float32)
                nrow = k * RC + rows
                for d in range(D):
                    col = plsc.load_gather(
                        n_buf, [nrow, jnp.full((L,), d, jnp.int32)])
                    cols.append(col)
                    ss += col * col
                    dt += col * wa1vec[d]
                sc = _renorm_scale(ss)
                e = ts + dt * sc
                e = jnp.maximum(e, 0.2 * e)
                if m is None:
                    m = e
                    S = jnp.full((L,), 1.0, jnp.float32)
                    for d in range(D):
                        agg[d] = sc * cols[d]
                else:
                    m_new = jnp.maximum(m, e)
                    r = jnp.exp(m - m_new)
                    p = jnp.exp(e - m_new)
                    S = S * r + p
                    w = p * sc
                    for d in range(D):
                        agg[d] = agg[d] * r + w * cols[d]
                    m = m_new

            gj = [jnp.zeros((L,), jnp.float32) for _ in range(NH)]
            for d in range(D):
                for j in range(NH):
                    gj[j] = gj[j] + agg[d] * wrows[j][d]

            ss_u = jnp.zeros((L,), jnp.float32)
            ucols = []
            for d in range(D):
                col = plsc.load_gather(
                    u_buf, [rows, jnp.full((L,), d, jnp.int32)])
                ss_u += col * col
                ucols.append(col)
            su = _renorm_scale(ss_u)
            uv = jnp.zeros((L,), jnp.float32)
            for j in range(NH):
                uv = uv + (ucols[j] + ucols[NH + j]) * gj[j]
            uv = uv * (su / S)
            out = 1.0 / (1.0 + jnp.exp(-uv))
            out_v[pl.ds(c * RC + g * L, L)] = out
            return carry2

        lax.fori_loop(0, NG, group, 0)

    fire(0, t_a, n_a, u_a, sem_a)

    def pair_body(t, carry):
        c0 = 2 * t
        c1 = c0 + 1
        fire(c1, t_b, n_b, u_b, sem_b)
        drain(t_a, n_a, u_a, sem_a)
        compute(c0, t_a, n_a, u_a)

        @pl.when(c1 + 1 < NCHUNK)
        def _():
            fire(c1 + 1, t_a, n_a, u_a, sem_a)

        drain(t_b, n_b, u_b, sem_b)
        compute(c1, t_b, n_b, u_b)
        return carry

    lax.fori_loop(0, NCHUNK // 2, pair_body, 0)
    pltpu.sync_copy(out_v, out_hbm.at[pl.ds(base, BPW)])


_sc_call = functools.partial(
    pl.kernel,
    out_type=jax.ShapeDtypeStruct((B,), jnp.float32),
    mesh=plsc.VectorSubcoreMesh(core_axis_name="c", subcore_axis_name="s"),
    compiler_params=pltpu.CompilerParams(
        needs_layout_passes=False, use_tc_tiling_on_sc=False),
    scratch_types=[
        pltpu.VMEM((BPW,), jnp.int32),
        pltpu.VMEM((BPW,), jnp.int32),
        pltpu.VMEM((BPW * K,), jnp.int32),
        pltpu.VMEM((RC, D), jnp.float32),
        pltpu.VMEM((RC * K, D), jnp.float32),
        pltpu.VMEM((RC, D), jnp.float32),
        pltpu.VMEM((RC, D), jnp.float32),
        pltpu.VMEM((RC * K, D), jnp.float32),
        pltpu.VMEM((RC, D), jnp.float32),
        pltpu.VMEM((NH * L,), jnp.float32),
        pltpu.VMEM((L,), jnp.float32),
        pltpu.VMEM((D,), jnp.float32),
        pltpu.VMEM((D,), jnp.float32),
        pltpu.VMEM((BPW,), jnp.float32),
        pltpu.SemaphoreType.DMA,
        pltpu.SemaphoreType.DMA,
        pltpu.SemaphoreType.DMA,
    ],
)(_body)


def kernel(u, i, neighbors, user_table, entity_table, W, a):
    u = u.astype(jnp.int32)
    i = i.astype(jnp.int32)
    nb = neighbors.astype(jnp.int32).T.reshape(-1)
    wt = W.T.reshape(-1)
    return _sc_call(u, i, nb, user_table, entity_table, wt, a.reshape(-1))
